# baseline (speedup 1.0000x reference)
import functools

import jax
import jax.numpy as jnp
from jax import lax
from jax.experimental import pallas as pl
from jax.experimental.pallas import tpu as pltpu
from jax.experimental.pallas import tpu_sc as plsc


def _gate_body(x_ref, wg_ref, bg_ref, idx_ref, rows_ref):
    N, E = idx_ref.shape[0], wg_ref.shape[1]
    logits = jnp.dot(x_ref[...], wg_ref[...], preferred_element_type=jnp.float32)
    logits = logits + bg_ref[...][None, :]
    m = jnp.max(logits, axis=-1, keepdims=True)
    p = jnp.exp(logits - m)
    p = p / jnp.sum(p, axis=-1, keepdims=True)
    pm = jnp.max(p, axis=-1, keepdims=True)
    lanes = jax.lax.broadcasted_iota(jnp.int32, p.shape, 1)
    idx = jnp.max(jnp.where(p >= pm, lanes, -1), axis=-1, keepdims=True)
    idx_ref[...] = idx
    col = jax.lax.broadcasted_iota(jnp.int32, (N, N), 1)
    rows_ref[...] = idx * N + col


def _expert_body(x_ref, we_ref, be_ref, tab_ref):
    e = pl.program_id(0)
    tab_ref[...] = (
        jnp.dot(x_ref[...], we_ref[0], preferred_element_type=jnp.float32)
        + be_ref[pl.ds(e, 1), :]
    )


def _make_sc_gather(B, D_FF, CHUNK):
    info = plsc.get_sparse_core_info()
    NC = info.num_cores
    NW = NC * info.num_subcores
    n_chunks = B // (NW * CHUNK)
    mesh = plsc.VectorSubcoreMesh(core_axis_name="c", subcore_axis_name="s")

    @functools.partial(
        pl.kernel,
        mesh=mesh,
        out_type=jax.ShapeDtypeStruct((B, D_FF), jnp.float32),
        scratch_types=[
            pltpu.VMEM((CHUNK,), jnp.int32),
            pltpu.VMEM((CHUNK, D_FF), jnp.float32),
            pltpu.SemaphoreType.DMA,
        ],
    )
    def sc_gather(tab_hbm, rows_hbm, out_hbm, idx_v, buf_v, sem):
        wid = lax.axis_index("s") * NC + lax.axis_index("c")

        def chunk_body(k, _):
            base = (wid * n_chunks + k) * CHUNK
            pltpu.sync_copy(rows_hbm.at[pl.ds(base, CHUNK)], idx_v)
            pltpu.async_copy(tab_hbm.at[idx_v], buf_v, sem).wait()
            pltpu.sync_copy(buf_v, out_hbm.at[pl.ds(base, CHUNK)])
            return 0

        lax.fori_loop(0, n_chunks, chunk_body, 0)

    return sc_gather


def kernel(x, W_gate, b_gate, W_experts, b_experts):
    N, D_MODEL = x.shape
    E = W_gate.shape[1]
    D_FF = W_experts.shape[2]
    B = N * N
    CHUNK = 128

    idx, rows = pl.pallas_call(
        _gate_body,
        out_shape=(
            jax.ShapeDtypeStruct((N, 1), jnp.int32),
            jax.ShapeDtypeStruct((N, N), jnp.int32),
        ),
    )(x, W_gate, b_gate)

    table = pl.pallas_call(
        _expert_body,
        grid=(E,),
        in_specs=[
            pl.BlockSpec((N, D_MODEL), lambda i: (0, 0)),
            pl.BlockSpec((1, D_MODEL, D_FF), lambda i: (i, 0, 0)),
            pl.BlockSpec((E, D_FF), lambda i: (0, 0)),
        ],
        out_specs=pl.BlockSpec((N, D_FF), lambda i: (i, 0)),
        out_shape=jax.ShapeDtypeStruct((E * N, D_FF), jnp.float32),
    )(x, W_experts, b_experts)

    sc_gather = _make_sc_gather(B, D_FF, CHUNK)
    out = sc_gather(table, rows.reshape(B))
    return out.reshape(N, N, D_FF)
